# baseline (speedup 1.0000x reference)
import jax
import jax.numpy as jnp
from jax.experimental import pallas as pl
from jax.experimental.pallas import tpu as pltpu

_BM = 512
_SUB = 512
_LANES = 128


def _km_kernel(f_ref, p_ref, sim_ref, ids_ref, phat_ref):
    i = pl.program_id(0)
    n = p_ref.shape[0]

    @pl.when(i == 0)
    def _norm_p():
        p = p_ref[...]
        pn = jnp.sqrt(jnp.sum(p * p, axis=1, keepdims=True))
        phat_ref[...] = p / jnp.maximum(pn, 1e-12)

    f = f_ref[...]
    fn = jnp.sqrt(jnp.sum(f * f, axis=1, keepdims=True))
    f = f / jnp.maximum(fn, 1e-12)

    sub_chunks = _SUB // _LANES
    amax = jnp.full((_BM, _LANES), -jnp.inf, dtype=jnp.float32)
    aidx = jnp.zeros((_BM, _LANES), dtype=jnp.int32)
    for s in range(n // _SUB):
        ps = phat_ref[pl.ds(s * _SUB, _SUB), :]
        sim_ref[:, s * _SUB:(s + 1) * _SUB] = jax.lax.dot_general(
            f, ps, (((1,), (1,)), ((), ())),
            preferred_element_type=jnp.float32)
        for k in range(sub_chunks):
            base = s * _SUB + k * _LANES
            vv = sim_ref[:, base:base + _LANES]
            chunk_id = s * sub_chunks + k
            gt = vv > amax
            amax = jnp.maximum(amax, vv)
            aidx = jnp.where(gt, chunk_id, aidx)

    rowmax = jnp.max(amax, axis=1, keepdims=True)
    lane = jax.lax.broadcasted_iota(jnp.int32, amax.shape, 1)
    col = aidx * _LANES + lane
    cand = jnp.where(amax == rowmax, col, jnp.iinfo(jnp.int32).max)
    ids_ref[...] = jnp.min(cand, axis=1, keepdims=True)


def kernel(features, prototypes):
    m, k = features.shape
    n = prototypes.shape[0]
    sim, ids = pl.pallas_call(
        _km_kernel,
        grid=(m // _BM,),
        in_specs=[
            pl.BlockSpec((_BM, k), lambda i: (i, 0)),
            pl.BlockSpec((n, k), lambda i: (0, 0)),
        ],
        out_specs=[
            pl.BlockSpec((_BM, n), lambda i: (i, 0)),
            pl.BlockSpec((_BM, 1), lambda i: (i, 0)),
        ],
        out_shape=[
            jax.ShapeDtypeStruct((m, n), jnp.float32),
            jax.ShapeDtypeStruct((m, 1), jnp.int32),
        ],
        scratch_shapes=[
            pltpu.VMEM((n, k), jnp.float32),
        ],
        compiler_params=pltpu.CompilerParams(
            dimension_semantics=("arbitrary",),
        ),
    )(features, prototypes)
    return ids.reshape(m), sim
